# Initial kernel scaffold; baseline (speedup 1.0000x reference)
import jax
import jax.numpy as jnp
from jax.experimental import pallas as pl


def kernel(x, rand_proj, values, codebook):
    raise NotImplementedError("write your pallas kernel here")


# speedup vs baseline: 1.1437x; 1.1437x over previous
import functools

import jax
import jax.numpy as jnp
from jax import lax
from jax.experimental import pallas as pl
from jax.experimental.pallas import tpu as pltpu
from jax.experimental.pallas import tpu_sc as plsc

B, N, DIM_EMBED = 8, 576, 384
DIM = 32
HEADS = 2
K = 8192
DIM_MEM = 32

TOKENS = B * N
T_BLK = 256
G = TOKENS // T_BLK

SC_CORES = 2
SC_SUBCORES = 16
SC_WORKERS = SC_CORES * SC_SUBCORES
TOK_PER_W = TOKENS // SC_WORKERS
GATHER_CHUNK = 72
N_CHUNKS = TOK_PER_W // GATHER_CHUNK


def _tc_idx_body(x_ref, rp_ref, cb_ref, idx_ref, cbn_ref):
    @pl.when(pl.program_id(0) == 0)
    def _():
        for h in range(HEADS):
            cb = cb_ref[h]
            cbn_ref[h] = jnp.sum(cb * cb, axis=-1)

    xb = x_ref[...]
    outs = []
    for h in range(HEADS):
        rp = rp_ref[h]
        cb = cb_ref[h]
        xp = jnp.dot(xb, rp)
        xp2 = jnp.sum(xp * xp, axis=-1, keepdims=True)
        dot = lax.dot_general(xp, cb, (((1,), (1,)), ((), ())))
        d2 = xp2 - 2.0 * dot + cbn_ref[h][None, :]
        m = jnp.min(d2, axis=-1, keepdims=True)
        ii = lax.broadcasted_iota(jnp.int32, d2.shape, 1)
        am = jnp.min(jnp.where(d2 == m, ii, K), axis=-1)
        outs.append(am + h * K)
    idx_ref[...] = jnp.stack(outs)[None]


def _tc_indices(xf, rand_proj, codebook):
    return pl.pallas_call(
        _tc_idx_body,
        grid=(G,),
        in_specs=[
            pl.BlockSpec((T_BLK, DIM_EMBED), lambda g: (g, 0)),
            pl.BlockSpec((HEADS, DIM_EMBED, DIM), lambda g: (0, 0, 0)),
            pl.BlockSpec((HEADS, K, DIM), lambda g: (0, 0, 0)),
        ],
        out_specs=pl.BlockSpec((1, HEADS, T_BLK), lambda g: (g, 0, 0)),
        out_shape=jax.ShapeDtypeStruct((G, HEADS, T_BLK), jnp.int32),
        scratch_shapes=[pltpu.VMEM((HEADS, K), jnp.float32)],
    )(xf, rand_proj, codebook)


@functools.partial(
    pl.kernel,
    mesh=plsc.VectorSubcoreMesh(core_axis_name="c", subcore_axis_name="s"),
    compiler_params=pltpu.CompilerParams(use_tc_tiling_on_sc=False),
    out_type=jax.ShapeDtypeStruct((TOKENS, DIM_MEM), jnp.float32),
    scratch_types=[
        pltpu.VMEM((N_CHUNKS, GATHER_CHUNK), jnp.int32),
        pltpu.VMEM((N_CHUNKS, GATHER_CHUNK), jnp.int32),
        pltpu.VMEM((TOK_PER_W, DIM_MEM), jnp.float32),
        pltpu.VMEM((TOK_PER_W, DIM_MEM), jnp.float32),
        pltpu.VMEM((TOK_PER_W, DIM_MEM), jnp.float32),
        pltpu.SemaphoreType.DMA,
    ],
)
def _sc_gather_mean(vals_hbm, idx0_hbm, idx1_hbm, out_hbm,
                    i0_v, i1_v, r0_v, r1_v, o_v, sem):
    wid = lax.axis_index("s") * SC_CORES + lax.axis_index("c")
    base = wid * TOK_PER_W
    pltpu.sync_copy(idx0_hbm.at[wid], i0_v)
    pltpu.sync_copy(idx1_hbm.at[wid], i1_v)
    copies = []
    for j in range(N_CHUNKS):
        sl = pl.ds(j * GATHER_CHUNK, GATHER_CHUNK)
        copies.append(pltpu.async_copy(vals_hbm.at[i0_v.at[j]], r0_v.at[sl], sem))
        copies.append(pltpu.async_copy(vals_hbm.at[i1_v.at[j]], r1_v.at[sl], sem))
    for c in copies:
        c.wait()

    def body(t, carry):
        for j in range(DIM_MEM // 16):
            sl = pl.ds(j * 16, 16)
            o_v[t, sl] = (r0_v[t, sl] + r1_v[t, sl]) * 0.5
        return carry

    lax.fori_loop(0, TOK_PER_W, body, 0)
    pltpu.sync_copy(o_v, out_hbm.at[pl.ds(base, TOK_PER_W)])


def kernel(x, rand_proj, values, codebook):
    xf = x.reshape(TOKENS, DIM_EMBED)
    idx = _tc_indices(xf, rand_proj, codebook)
    idxh = idx.transpose(1, 0, 2).reshape(HEADS, SC_WORKERS, N_CHUNKS, GATHER_CHUNK)
    vals_flat = values.reshape(HEADS * K, DIM_MEM)
    out = _sc_gather_mean(vals_flat, idxh[0], idxh[1])
    return out.reshape(B, N, DIM_MEM)


# speedup vs baseline: 1.4198x; 1.2415x over previous
import functools

import jax
import jax.numpy as jnp
from jax import lax
from jax.experimental import pallas as pl
from jax.experimental.pallas import tpu as pltpu
from jax.experimental.pallas import tpu_sc as plsc

B, N, DIM_EMBED = 8, 576, 384
DIM = 32
HEADS = 2
K = 8192
DIM_MEM = 32

TOKENS = B * N
T_BLK = 256
G = TOKENS // T_BLK

SC_CORES = 2
SC_SUBCORES = 16
SC_WORKERS = SC_CORES * SC_SUBCORES
TOK_PER_W = TOKENS // SC_WORKERS
GATHER_CHUNK = 72
N_CHUNKS = TOK_PER_W // GATHER_CHUNK


def _tc_idx_body(x_ref, rp_ref, cb_ref, cbn_ref, idx_ref):
    xb = x_ref[...]
    outs = []
    for h in range(HEADS):
        rp = rp_ref[h]
        cb = cb_ref[h]
        xp = jnp.dot(xb, rp)
        xp2 = jnp.sum(xp * xp, axis=-1, keepdims=True)
        dotn = lax.dot_general(xp * -2.0, cb, (((1,), (1,)), ((), ())))
        d2 = xp2 + dotn + cbn_ref[h][None, :]
        am = jnp.argmin(d2, axis=-1).astype(jnp.int32)
        outs.append(am + h * K)
    idx_ref[...] = jnp.stack(outs)[None]


def _tc_indices(xf, rand_proj, codebook, cbn):
    return pl.pallas_call(
        _tc_idx_body,
        grid=(G,),
        in_specs=[
            pl.BlockSpec((T_BLK, DIM_EMBED), lambda g: (g, 0)),
            pl.BlockSpec((HEADS, DIM_EMBED, DIM), lambda g: (0, 0, 0)),
            pl.BlockSpec((HEADS, K, DIM), lambda g: (0, 0, 0)),
            pl.BlockSpec((HEADS, K), lambda g: (0, 0)),
        ],
        out_specs=pl.BlockSpec((1, HEADS, T_BLK), lambda g: (g, 0, 0)),
        out_shape=jax.ShapeDtypeStruct((G, HEADS, T_BLK), jnp.int32),
    )(xf, rand_proj, codebook, cbn)


@functools.partial(
    pl.kernel,
    mesh=plsc.VectorSubcoreMesh(core_axis_name="c", subcore_axis_name="s"),
    compiler_params=pltpu.CompilerParams(use_tc_tiling_on_sc=False),
    out_type=jax.ShapeDtypeStruct((TOKENS, DIM_MEM), jnp.float32),
    scratch_types=[
        pltpu.VMEM((N_CHUNKS, GATHER_CHUNK), jnp.int32),
        pltpu.VMEM((N_CHUNKS, GATHER_CHUNK), jnp.int32),
        pltpu.VMEM((TOK_PER_W, DIM_MEM), jnp.float32),
        pltpu.VMEM((TOK_PER_W, DIM_MEM), jnp.float32),
        pltpu.VMEM((TOK_PER_W, DIM_MEM), jnp.float32),
        pltpu.SemaphoreType.DMA,
    ],
)
def _sc_gather_mean(vals_hbm, idx0_hbm, idx1_hbm, out_hbm,
                    i0_v, i1_v, r0_v, r1_v, o_v, sem):
    wid = lax.axis_index("s") * SC_CORES + lax.axis_index("c")
    base = wid * TOK_PER_W
    pltpu.sync_copy(idx0_hbm.at[wid], i0_v)
    pltpu.sync_copy(idx1_hbm.at[wid], i1_v)
    copies = []
    for j in range(N_CHUNKS):
        sl = pl.ds(j * GATHER_CHUNK, GATHER_CHUNK)
        copies.append(pltpu.async_copy(vals_hbm.at[i0_v.at[j]], r0_v.at[sl], sem))
        copies.append(pltpu.async_copy(vals_hbm.at[i1_v.at[j]], r1_v.at[sl], sem))
    for c in copies:
        c.wait()

    def body(t, carry):
        for j in range(DIM_MEM // 16):
            sl = pl.ds(j * 16, 16)
            o_v[t, sl] = (r0_v[t, sl] + r1_v[t, sl]) * 0.5
        return carry

    lax.fori_loop(0, TOK_PER_W, body, 0)
    pltpu.sync_copy(o_v, out_hbm.at[pl.ds(base, TOK_PER_W)])


def kernel(x, rand_proj, values, codebook):
    xf = x.reshape(TOKENS, DIM_EMBED)
    cbn = jnp.sum(codebook * codebook, axis=-1)
    idx = _tc_indices(xf, rand_proj, codebook, cbn)
    idxh = idx.transpose(1, 0, 2).reshape(HEADS, SC_WORKERS, N_CHUNKS, GATHER_CHUNK)
    vals_flat = values.reshape(HEADS * K, DIM_MEM)
    out = _sc_gather_mean(vals_flat, idxh[0], idxh[1])
    return out.reshape(B, N, DIM_MEM)
